# baseline (speedup 1.0000x reference)
import functools

import jax
import jax.numpy as jnp
from jax import lax
from jax.experimental import pallas as pl
from jax.experimental.pallas import tpu as pltpu
from jax.experimental.pallas import tpu_sc as plsc

VOCAB = 100000
HIDDEN = 2048
B = 8192

NUM_CORES = 2
NUM_SUBCORES = 16
NW = NUM_CORES * NUM_SUBCORES
BPW = B // NW
SROWS = 32


def _emb_kernel(idx_hbm, table_hbm, out_hbm, idx_v, stage_sh, dsem, osem):
    sid = lax.axis_index("s")
    wid = sid * NUM_CORES + lax.axis_index("c")
    base = wid * BPW
    pltpu.sync_copy(idx_hbm.at[pl.ds(base, BPW)], idx_v)

    def pass_body(p, _):
        off = p * SROWS

        def grp_body(g, _):
            rows16 = idx_v[pl.ds(off + g * 16, 16)]
            handles = []
            for i in range(16):
                row = jax.lax.squeeze(jax.lax.slice(rows16, (i,), (i + 1,)), (0,))
                handles.append(pltpu.async_copy(
                    table_hbm.at[pl.ds(row, 1)],
                    stage_sh.at[sid, pl.ds(g * 16 + i, 1)],
                    dsem,
                ))
            for h in handles:
                h.wait()
            return 0

        lax.fori_loop(0, SROWS // 16, grp_body, 0)
        pltpu.async_copy(
            stage_sh.at[sid],
            out_hbm.at[pl.ds(base + off, SROWS)],
            osem,
        ).wait()
        return 0

    lax.fori_loop(0, BPW // SROWS, pass_body, 0)


@jax.jit
def _emb(idx_flat, table):
    mesh = plsc.VectorSubcoreMesh(core_axis_name="c", subcore_axis_name="s")
    f = functools.partial(
        pl.kernel,
        mesh=mesh,
        out_type=jax.ShapeDtypeStruct((B, HIDDEN), jnp.float32),
        scratch_types=[
            pltpu.VMEM((BPW,), jnp.int32),
            pltpu.VMEM_SHARED((NUM_SUBCORES, SROWS, HIDDEN), jnp.float32),
            pltpu.SemaphoreType.DMA,
            pltpu.SemaphoreType.DMA,
        ],
    )(_emb_kernel)
    return f(idx_flat, table)


def kernel(token_ids, embed_weight):
    batch, seq = token_ids.shape
    idx_flat = token_ids.reshape(-1).astype(jnp.int32)
    out = _emb(idx_flat, embed_weight)
    return out.reshape(batch, seq, HIDDEN)


# speedup vs baseline: 1.0156x; 1.0156x over previous
import functools

import jax
import jax.numpy as jnp
from jax import lax
from jax.experimental import pallas as pl
from jax.experimental.pallas import tpu as pltpu
from jax.experimental.pallas import tpu_sc as plsc

VOCAB = 100000
HIDDEN = 2048
BATCH = 4
SEQ = 2048
B = BATCH * SEQ

NUM_CORES = 2
NUM_SUBCORES = 16
NW = NUM_CORES * NUM_SUBCORES
BPW = B // NW
CS = 16
DS = 16
K = BPW // (CS + DS)
STREAM_ROWS = CS * K
WPB = SEQ // BPW


def _emb_kernel(idx_hbm, table_hbm, out_hbm, idx_v, rows_v, stage_sh,
                gsem, ssem, dsem, osem):
    sid = lax.axis_index("s")
    wid = sid * NUM_CORES + lax.axis_index("c")
    b0 = wid // WPB
    col = (wid % WPB) * BPW
    pltpu.sync_copy(idx_hbm.at[b0, pl.ds(col, BPW)], idx_v)

    def body(k, _):
        buf = lax.rem(k, 2)
        s_off = k * CS
        d_off = STREAM_ROWS + k * DS
        gs = pltpu.async_copy(
            table_hbm.at[idx_v.at[pl.ds(s_off, CS)]],
            rows_v.at[buf],
            gsem,
        )
        rows16 = idx_v[pl.ds(d_off, DS)]
        hs = []
        for i in range(DS):
            row = lax.squeeze(lax.slice(rows16, (i,), (i + 1,)), (0,))
            hs.append(pltpu.async_copy(
                table_hbm.at[pl.ds(row, 1)],
                stage_sh.at[sid, pl.ds(i, 1)],
                dsem,
            ))
        gs.wait()
        ss = pltpu.async_copy(
            rows_v.at[buf],
            out_hbm.at[b0, pl.ds(col + s_off, CS)],
            ssem,
        )
        for h in hs:
            h.wait()
        wo = pltpu.async_copy(
            stage_sh.at[sid],
            out_hbm.at[b0, pl.ds(col + d_off, DS)],
            osem,
        )
        ss.wait()
        wo.wait()
        return 0

    lax.fori_loop(0, K, body, 0)


@jax.jit
def _emb(token_ids, table):
    mesh = plsc.VectorSubcoreMesh(core_axis_name="c", subcore_axis_name="s")
    f = functools.partial(
        pl.kernel,
        mesh=mesh,
        out_type=jax.ShapeDtypeStruct((BATCH, SEQ, HIDDEN), jnp.float32),
        scratch_types=[
            pltpu.VMEM((BPW,), jnp.int32),
            pltpu.VMEM((2, CS, HIDDEN), jnp.float32),
            pltpu.VMEM_SHARED((NUM_SUBCORES, DS, HIDDEN), jnp.float32),
            pltpu.SemaphoreType.DMA,
            pltpu.SemaphoreType.DMA,
            pltpu.SemaphoreType.DMA,
            pltpu.SemaphoreType.DMA,
        ],
    )(_emb_kernel)
    return f(token_ids, table)


def kernel(token_ids, embed_weight):
    return _emb(token_ids.astype(jnp.int32), embed_weight)


# speedup vs baseline: 1.0303x; 1.0145x over previous
import functools

import jax
import jax.numpy as jnp
from jax import lax
from jax.experimental import pallas as pl
from jax.experimental.pallas import tpu as pltpu
from jax.experimental.pallas import tpu_sc as plsc

VOCAB = 100000
HIDDEN = 2048
BATCH = 4
SEQ = 2048
B = BATCH * SEQ

NUM_CORES = 2
NUM_SUBCORES = 16
NW = NUM_CORES * NUM_SUBCORES
BPW = B // NW
CHUNK = 16
NCHUNK = BPW // CHUNK
WPB = SEQ // BPW


def _emb_kernel(idx_hbm, table_hbm, out_hbm, idx_v, rows_v, gsem, ssem):
    wid = lax.axis_index("s") * NUM_CORES + lax.axis_index("c")
    b0 = wid // WPB
    col = (wid % WPB) * BPW
    pltpu.sync_copy(idx_hbm.at[b0, pl.ds(col, BPW)], idx_v)

    def issue_g(ch, buf):
        return pltpu.async_copy(
            table_hbm.at[idx_v.at[pl.ds(ch * CHUNK, CHUNK)]],
            rows_v.at[buf],
            gsem,
        )

    def issue_s(ch, buf):
        return pltpu.async_copy(
            rows_v.at[buf],
            out_hbm.at[b0, pl.ds(col + ch * CHUNK, CHUNK)],
            ssem,
        )

    def body(k, _):
        c0 = 2 * k
        c1 = 2 * k + 1
        g0 = issue_g(c0, 0)
        g1 = issue_g(c1, 1)
        g0.wait()
        s0 = issue_s(c0, 0)
        g1.wait()
        s1 = issue_s(c1, 1)
        s0.wait()
        s1.wait()
        return 0

    lax.fori_loop(0, NCHUNK // 2, body, 0)


@jax.jit
def _emb(token_ids, table):
    mesh = plsc.VectorSubcoreMesh(core_axis_name="c", subcore_axis_name="s")
    f = functools.partial(
        pl.kernel,
        mesh=mesh,
        out_type=jax.ShapeDtypeStruct((BATCH, SEQ, HIDDEN), jnp.float32),
        scratch_types=[
            pltpu.VMEM((BPW,), jnp.int32),
            pltpu.VMEM((2, CHUNK, HIDDEN), jnp.float32),
            pltpu.SemaphoreType.DMA,
            pltpu.SemaphoreType.DMA,
        ],
    )(_emb_kernel)
    return f(token_ids, table)


def kernel(token_ids, embed_weight):
    return _emb(token_ids.astype(jnp.int32), embed_weight)
